# baseline (speedup 1.0000x reference)
import functools

import jax
import jax.numpy as jnp
from jax import lax
from jax.experimental import pallas as pl
from jax.experimental.pallas import tpu as pltpu
from jax.experimental.pallas import tpu_sc as plsc


def _build(B, C, H, W, Hr, Wr):
    ph = (Hr - H) // 2
    pw = (Wr - W) // 2
    C2 = 2 * C

    NC, NS = 2, 16
    NW = NC * NS
    PZ = (B * C) // NW
    ZB = 2

    mesh = plsc.VectorSubcoreMesh(core_axis_name="c", subcore_axis_name="s")

    @functools.partial(
        pl.kernel,
        mesh=mesh,
        out_type=jax.ShapeDtypeStruct((B, C2, Hr, Wr), jnp.float32),
        scratch_types=[
            pltpu.VMEM((ZB, Hr, Wr), jnp.float32),
            pltpu.SemaphoreType.DMA,
        ],
        compiler_params=pltpu.CompilerParams(use_tc_tiling_on_sc=True),
    )
    def fill_zero_planes(zp_hbm, out_hbm, zbuf, sem):
        wid = lax.axis_index("s") * NC + lax.axis_index("c")
        pltpu.sync_copy(zp_hbm, zbuf)
        handles = []
        for i in range(PZ // ZB):
            p = wid * PZ + i * ZB
            b = p // C
            c = p % C
            handles.append(pltpu.async_copy(
                zbuf, out_hbm.at[b, pl.ds(C + c, ZB)], sem))
            handles.append(pltpu.async_copy(
                zbuf.at[:, pl.ds(0, ph)],
                out_hbm.at[b, pl.ds(c, ZB), pl.ds(0, ph)], sem))
            handles.append(pltpu.async_copy(
                zbuf.at[:, pl.ds(0, Hr - ph - H)],
                out_hbm.at[b, pl.ds(c, ZB), pl.ds(ph + H, Hr - ph - H)], sem))
        for h in handles:
            h.wait()

    CB = 32
    RB = ph

    def pad_body(x_ref, buf_ref, out_ref):
        del buf_ref
        out_ref[0, :, :, :pw] = jnp.zeros((CB, RB, pw), jnp.float32)
        out_ref[0, :, :, pw + W:] = jnp.zeros((CB, RB, Wr - pw - W), jnp.float32)
        out_ref[0, :, :, pw:pw + W] = x_ref[0]

    pad_x = pl.pallas_call(
        pad_body,
        grid=(B, C // CB, H // RB),
        in_specs=[
            pl.BlockSpec((1, CB, RB, W), lambda b, g, r: (b, g, r, 0)),
            pl.BlockSpec(memory_space=pl.ANY),
        ],
        out_specs=pl.BlockSpec((1, CB, RB, Wr), lambda b, g, r: (b, g, r + 1, 0)),
        out_shape=jax.ShapeDtypeStruct((B, C2, Hr, Wr), jnp.float32),
        input_output_aliases={1: 0},
    )

    def run(x):
        zp = jnp.zeros((ZB, Hr, Wr), jnp.float32)
        buf = fill_zero_planes(zp)
        return pad_x(x, buf)

    return run


def kernel(x, reg):
    B, C, H, W = x.shape
    Hr, Wr = reg.shape[2], reg.shape[3]
    return _build(B, C, H, W, Hr, Wr)(x)


# speedup vs baseline: 1.6051x; 1.6051x over previous
import functools

import jax
import jax.numpy as jnp
from jax import lax
from jax.experimental import pallas as pl
from jax.experimental.pallas import tpu as pltpu
from jax.experimental.pallas import tpu_sc as plsc


def _build(B, C, H, W, Hr, Wr):
    ph = (Hr - H) // 2
    pw = (Wr - W) // 2
    C2 = 2 * C

    NC, NS = 2, 16
    NW = NC * NS
    PZ = (B * C) // NW
    ZB = 2

    mesh = plsc.VectorSubcoreMesh(core_axis_name="c", subcore_axis_name="s")

    @functools.partial(
        pl.kernel,
        mesh=mesh,
        out_type=jax.ShapeDtypeStruct((B, C2, Hr, Wr), jnp.float32),
        scratch_types=[
            pltpu.VMEM((ZB, Hr, Wr), jnp.float32),
            pltpu.SemaphoreType.DMA,
        ],
        compiler_params=pltpu.CompilerParams(use_tc_tiling_on_sc=True),
    )
    def fill_zero_planes(zp_hbm, out_hbm, zbuf, sem):
        wid = lax.axis_index("s") * NC + lax.axis_index("c")
        pltpu.sync_copy(zp_hbm, zbuf)
        handles = []
        for i in range(PZ // ZB):
            p = wid * PZ + i * ZB
            b = p // C
            c = p % C
            handles.append(pltpu.async_copy(
                zbuf, out_hbm.at[b, pl.ds(C + c, ZB)], sem))
        for h in handles:
            h.wait()

    CB = 48

    def pad_body(x_ref, buf_ref, out_ref):
        del buf_ref
        out_ref[0, :, :ph, :] = jnp.zeros((CB, ph, Wr), jnp.float32)
        out_ref[0, :, ph + H:, :] = jnp.zeros((CB, Hr - ph - H, Wr), jnp.float32)
        out_ref[0, :, ph:ph + H, :pw] = jnp.zeros((CB, H, pw), jnp.float32)
        out_ref[0, :, ph:ph + H, pw + W:] = jnp.zeros((CB, H, Wr - pw - W), jnp.float32)
        out_ref[0, :, ph:ph + H, pw:pw + W] = x_ref[0]

    pad_x = pl.pallas_call(
        pad_body,
        grid=(B, C // CB),
        in_specs=[
            pl.BlockSpec((1, CB, H, W), lambda b, g: (b, g, 0, 0)),
            pl.BlockSpec(memory_space=pl.ANY),
        ],
        out_specs=pl.BlockSpec((1, CB, Hr, Wr), lambda b, g: (b, g, 0, 0)),
        out_shape=jax.ShapeDtypeStruct((B, C2, Hr, Wr), jnp.float32),
        input_output_aliases={1: 0},
    )

    def run(x):
        zp = jnp.zeros((ZB, Hr, Wr), jnp.float32)
        buf = fill_zero_planes(zp)
        return pad_x(x, buf)

    return run


def kernel(x, reg):
    B, C, H, W = x.shape
    Hr, Wr = reg.shape[2], reg.shape[3]
    return _build(B, C, H, W, Hr, Wr)(x)
